# Initial kernel scaffold; baseline (speedup 1.0000x reference)
import jax
import jax.numpy as jnp
from jax.experimental import pallas as pl


def kernel(src_words, trg_words, wmasks, labels, u_embeddings, v_embeddings):
    raise NotImplementedError("write your pallas kernel here")


# speedup vs baseline: 2.8450x; 2.8450x over previous
import functools

import jax
import jax.numpy as jnp
from jax import lax
from jax.experimental import pallas as pl
from jax.experimental.pallas import tpu as pltpu
from jax.experimental.pallas import tpu_sc as plsc

B = 4096
CTX = 20
K = 20
D = 128
LANES = 16

NW = 32
PER_W = B // NW
G = 4
R = G * CTX
NCH = PER_W // G
DC = D // LANES
NGRP = (G * K) // LANES


def _xs(v, o):
    perm = lax.iota(jnp.int32, LANES) ^ o
    return jnp.take_along_axis(v, perm, axis=0)


def _butterfly16(vecs):
    iota = lax.iota(jnp.int32, LANES)
    o = 1
    while len(vecs) > 1:
        m = (iota & o) == 0
        vecs = [jnp.where(m, a + _xs(a, o), b + _xs(b, o))
                for a, b in zip(vecs[::2], vecs[1::2])]
        o *= 2
    return vecs[0]


def _sc_pred_kernel(src_idx, trg_idx, u_emb, v_emb, out,
                    idx_u, idx_v, urows, vrows, srcacc, pred, sem_u, sem_v):
    wid = lax.axis_index("s") * 2 + lax.axis_index("c")

    pltpu.sync_copy(src_idx.at[wid], idx_u)
    pltpu.sync_copy(trg_idx.at[wid], idx_v)

    def chunk_body(j, carry):
        cu = pltpu.async_copy(u_emb.at[idx_u.at[j]], urows, sem_u)
        cv = pltpu.async_copy(v_emb.at[idx_v.at[j]], vrows, sem_v)
        cu.wait()
        cv.wait()
        for e in range(G):
            base = e * CTX
            for dc in range(DC):
                off = dc * LANES
                acc = urows[base, pl.ds(off, LANES)]
                for c in range(1, CTX):
                    acc = acc + urows[base + c, pl.ds(off, LANES)]
                srcacc[e, pl.ds(off, LANES)] = acc
        for g in range(NGRP):
            t0 = g * LANES
            partials = [jnp.zeros((LANES,), jnp.float32)] * LANES
            for dc in range(DC):
                off = dc * LANES
                ea = t0 // K
                eb = (t0 + LANES - 1) // K
                la = srcacc[ea, pl.ds(off, LANES)]
                lb = la if eb == ea else srcacc[eb, pl.ds(off, LANES)]
                for i in range(LANES):
                    t = t0 + i
                    a = la if t // K == ea else lb
                    partials[i] = partials[i] + a * vrows[t, pl.ds(off, LANES)]
            pred[j, pl.ds(t0, LANES)] = _butterfly16(partials)
        return carry

    lax.fori_loop(0, NCH, chunk_body, 0)
    pltpu.sync_copy(pred, out.at[wid])


def _sc_pred(src_words, trg_words, u_embeddings, v_embeddings):
    mesh = plsc.VectorSubcoreMesh(core_axis_name="c", subcore_axis_name="s")
    kern = functools.partial(
        pl.kernel,
        mesh=mesh,
        out_type=jax.ShapeDtypeStruct((NW, NCH, R), jnp.float32),
        scratch_types=[
            pltpu.VMEM((NCH, R), jnp.int32),
            pltpu.VMEM((NCH, R), jnp.int32),
            pltpu.VMEM((R, D), jnp.float32),
            pltpu.VMEM((R, D), jnp.float32),
            pltpu.VMEM((G, D), jnp.float32),
            pltpu.VMEM((NCH, R), jnp.float32),
            pltpu.SemaphoreType.DMA,
            pltpu.SemaphoreType.DMA,
        ],
    )(_sc_pred_kernel)
    src_idx = src_words.reshape(NW, NCH, R)
    trg_idx = trg_words.reshape(NW, NCH, R)
    return kern(src_idx, trg_idx, u_embeddings, v_embeddings)


def _loss_kernel(pred_ref, w_ref, y_ref, out_ref):
    p = pred_ref[...]
    w = w_ref[...]
    y = y_ref[...]
    bce = jnp.maximum(p, 0.0) - p * y + jnp.log1p(jnp.exp(-jnp.abs(p)))
    wl = w * bce
    num = jnp.sum(wl, axis=1)
    den = jnp.sum(w, axis=1)
    out_ref[0, 0] = jnp.mean(num / den)


def kernel(src_words, trg_words, wmasks, labels, u_embeddings, v_embeddings):
    pred = _sc_pred(src_words, trg_words, u_embeddings, v_embeddings)
    pred = pred.reshape(B, K)
    loss = pl.pallas_call(
        _loss_kernel,
        out_shape=jax.ShapeDtypeStruct((1, 1), jnp.float32),
        out_specs=pl.BlockSpec(memory_space=pltpu.SMEM),
    )(pred, wmasks, labels)
    return loss.reshape(())


# speedup vs baseline: 3.2369x; 1.1378x over previous
import functools

import jax
import jax.numpy as jnp
from jax import lax
from jax.experimental import pallas as pl
from jax.experimental.pallas import tpu as pltpu
from jax.experimental.pallas import tpu_sc as plsc

B = 4096
CTX = 20
K = 20
D = 128
LANES = 16

NW = 32
PER_W = B // NW
G = 4
R = G * CTX
NCH = PER_W // G
DC = D // LANES
NGRP = (G * K) // LANES


def _xs(v, o):
    perm = lax.iota(jnp.int32, LANES) ^ o
    return jnp.take_along_axis(v, perm, axis=0)


def _butterfly16(vecs):
    iota = lax.iota(jnp.int32, LANES)
    o = 1
    while len(vecs) > 1:
        m = (iota & o) == 0
        vecs = [jnp.where(m, a + _xs(a, o), b + _xs(b, o))
                for a, b in zip(vecs[::2], vecs[1::2])]
        o *= 2
    return vecs[0]


def _compute_chunk(j, urows, vrows, srcacc, pred):
    for e in range(G):
        base = e * CTX
        for dc in range(DC):
            off = dc * LANES
            acc = urows[base, pl.ds(off, LANES)]
            for c in range(1, CTX):
                acc = acc + urows[base + c, pl.ds(off, LANES)]
            srcacc[e, pl.ds(off, LANES)] = acc
    for g in range(NGRP):
        t0 = g * LANES
        partials = [jnp.zeros((LANES,), jnp.float32)] * LANES
        for dc in range(DC):
            off = dc * LANES
            ea = t0 // K
            eb = (t0 + LANES - 1) // K
            la = srcacc[ea, pl.ds(off, LANES)]
            lb = la if eb == ea else srcacc[eb, pl.ds(off, LANES)]
            for i in range(LANES):
                a = la if (t0 + i) // K == ea else lb
                partials[i] = partials[i] + a * vrows[t0 + i, pl.ds(off, LANES)]
        pred[j, pl.ds(t0, LANES)] = _butterfly16(partials)


def _sc_pred_kernel(src_idx, trg_idx, u_emb, v_emb, out,
                    idx_u, idx_v, u0, v0, u1, v1, srcacc, pred,
                    su0, sv0, su1, sv1):
    wid = lax.axis_index("s") * 2 + lax.axis_index("c")

    pltpu.sync_copy(src_idx.at[wid], idx_u)
    pltpu.sync_copy(trg_idx.at[wid], idx_v)

    def issue(jc, ub, vb, su, sv):
        pltpu.async_copy(u_emb.at[idx_u.at[jc]], ub, su)
        pltpu.async_copy(v_emb.at[idx_v.at[jc]], vb, sv)

    def wait(ub, vb, su, sv):
        pltpu.make_async_copy(u_emb.at[pl.ds(0, R)], ub, su).wait()
        pltpu.make_async_copy(v_emb.at[pl.ds(0, R)], vb, sv).wait()

    issue(0, u0, v0, su0, sv0)

    def pair_body(i, carry):
        j0 = 2 * i
        issue(j0 + 1, u1, v1, su1, sv1)
        wait(u0, v0, su0, sv0)
        _compute_chunk(j0, u0, v0, srcacc, pred)
        issue(jnp.minimum(j0 + 2, NCH - 1), u0, v0, su0, sv0)
        wait(u1, v1, su1, sv1)
        _compute_chunk(j0 + 1, u1, v1, srcacc, pred)
        return carry

    lax.fori_loop(0, NCH // 2, pair_body, 0)
    wait(u0, v0, su0, sv0)
    pltpu.sync_copy(pred, out.at[wid])


def _sc_pred(src_words, trg_words, u_embeddings, v_embeddings):
    mesh = plsc.VectorSubcoreMesh(core_axis_name="c", subcore_axis_name="s")
    kern = functools.partial(
        pl.kernel,
        mesh=mesh,
        out_type=jax.ShapeDtypeStruct((NW, NCH, R), jnp.float32),
        scratch_types=[
            pltpu.VMEM((NCH, R), jnp.int32),
            pltpu.VMEM((NCH, R), jnp.int32),
            pltpu.VMEM((R, D), jnp.float32),
            pltpu.VMEM((R, D), jnp.float32),
            pltpu.VMEM((R, D), jnp.float32),
            pltpu.VMEM((R, D), jnp.float32),
            pltpu.VMEM((G, D), jnp.float32),
            pltpu.VMEM((NCH, R), jnp.float32),
            pltpu.SemaphoreType.DMA,
            pltpu.SemaphoreType.DMA,
            pltpu.SemaphoreType.DMA,
            pltpu.SemaphoreType.DMA,
        ],
    )(_sc_pred_kernel)
    src_idx = src_words.reshape(NW, NCH, R)
    trg_idx = trg_words.reshape(NW, NCH, R)
    return kern(src_idx, trg_idx, u_embeddings, v_embeddings)


def _loss_kernel(pred_ref, w_ref, y_ref, out_ref):
    p = pred_ref[...]
    w = w_ref[...]
    y = y_ref[...]
    bce = jnp.maximum(p, 0.0) - p * y + jnp.log1p(jnp.exp(-jnp.abs(p)))
    wl = w * bce
    num = jnp.sum(wl, axis=1)
    den = jnp.sum(w, axis=1)
    out_ref[0, 0] = jnp.mean(num / den)


def kernel(src_words, trg_words, wmasks, labels, u_embeddings, v_embeddings):
    pred = _sc_pred(src_words, trg_words, u_embeddings, v_embeddings)
    pred = pred.reshape(B, K)
    loss = pl.pallas_call(
        _loss_kernel,
        out_shape=jax.ShapeDtypeStruct((1, 1), jnp.float32),
        out_specs=pl.BlockSpec(memory_space=pltpu.SMEM),
    )(pred, wmasks, labels)
    return loss.reshape(())


# speedup vs baseline: 3.2800x; 1.0133x over previous
import functools

import jax
import jax.numpy as jnp
from jax import lax
from jax.experimental import pallas as pl
from jax.experimental.pallas import tpu as pltpu
from jax.experimental.pallas import tpu_sc as plsc

B = 4096
CTX = 20
K = 20
D = 128
LANES = 16

NW = 32
PER_W = B // NW
G = 4
R = G * CTX
NCH = PER_W // G
DC = D // LANES
NGRP = (G * K) // LANES


def _xs(v, o):
    perm = lax.iota(jnp.int32, LANES) ^ o
    return jnp.take_along_axis(v, perm, axis=0)


def _butterfly16(vecs):
    iota = lax.iota(jnp.int32, LANES)
    o = 1
    while len(vecs) > 1:
        m = (iota & o) == 0
        vecs = [jnp.where(m, a + _xs(a, o), b + _xs(b, o))
                for a, b in zip(vecs[::2], vecs[1::2])]
        o *= 2
    return vecs[0]


def _compute_chunk(j, urows, vrows, srcacc, pred):
    for e in range(G):
        base = e * CTX
        for dc in range(DC):
            off = dc * LANES
            acc = urows[base, pl.ds(off, LANES)]
            for c in range(1, CTX):
                acc = acc + urows[base + c, pl.ds(off, LANES)]
            srcacc[e, pl.ds(off, LANES)] = acc
    for g in range(NGRP):
        t0 = g * LANES
        partials = [jnp.zeros((LANES,), jnp.float32)] * LANES
        for dc in range(DC):
            off = dc * LANES
            ea = t0 // K
            eb = (t0 + LANES - 1) // K
            la = srcacc[ea, pl.ds(off, LANES)]
            lb = la if eb == ea else srcacc[eb, pl.ds(off, LANES)]
            for i in range(LANES):
                a = la if (t0 + i) // K == ea else lb
                partials[i] = partials[i] + a * vrows[t0 + i, pl.ds(off, LANES)]
        pred[j, pl.ds(t0, LANES)] = _butterfly16(partials)


H = R // 2


def _sc_pred_kernel(src_idx, trg_idx, u_emb, v_emb, out,
                    idx_u, idx_v, u0, v0, u1, v1, srcacc, pred,
                    su0, sv0, su1, sv1, tu0, tv0, tu1, tv1):
    wid = lax.axis_index("s") * 2 + lax.axis_index("c")

    pltpu.sync_copy(src_idx.at[wid], idx_u)
    pltpu.sync_copy(trg_idx.at[wid], idx_v)

    def issue(jc, ub, vb, su, sv, tu, tv):
        pltpu.async_copy(u_emb.at[idx_u.at[jc, pl.ds(0, H)]], ub.at[pl.ds(0, H)], su)
        pltpu.async_copy(v_emb.at[idx_v.at[jc, pl.ds(0, H)]], vb.at[pl.ds(0, H)], sv)
        pltpu.async_copy(u_emb.at[idx_u.at[jc, pl.ds(H, H)]], ub.at[pl.ds(H, H)], tu)
        pltpu.async_copy(v_emb.at[idx_v.at[jc, pl.ds(H, H)]], vb.at[pl.ds(H, H)], tv)

    def wait(ub, vb, su, sv, tu, tv):
        pltpu.make_async_copy(u_emb.at[pl.ds(0, H)], ub.at[pl.ds(0, H)], su).wait()
        pltpu.make_async_copy(v_emb.at[pl.ds(0, H)], vb.at[pl.ds(0, H)], sv).wait()
        pltpu.make_async_copy(u_emb.at[pl.ds(0, H)], ub.at[pl.ds(H, H)], tu).wait()
        pltpu.make_async_copy(v_emb.at[pl.ds(0, H)], vb.at[pl.ds(H, H)], tv).wait()

    issue(0, u0, v0, su0, sv0, tu0, tv0)

    def pair_body(i, carry):
        j0 = 2 * i
        issue(j0 + 1, u1, v1, su1, sv1, tu1, tv1)
        wait(u0, v0, su0, sv0, tu0, tv0)
        _compute_chunk(j0, u0, v0, srcacc, pred)
        issue(jnp.minimum(j0 + 2, NCH - 1), u0, v0, su0, sv0, tu0, tv0)
        wait(u1, v1, su1, sv1, tu1, tv1)
        _compute_chunk(j0 + 1, u1, v1, srcacc, pred)
        return carry

    lax.fori_loop(0, NCH // 2, pair_body, 0)
    wait(u0, v0, su0, sv0, tu0, tv0)
    pltpu.sync_copy(pred, out.at[wid])


def _sc_pred(src_words, trg_words, u_embeddings, v_embeddings):
    mesh = plsc.VectorSubcoreMesh(core_axis_name="c", subcore_axis_name="s")
    kern = functools.partial(
        pl.kernel,
        mesh=mesh,
        out_type=jax.ShapeDtypeStruct((NW, NCH, R), jnp.float32),
        scratch_types=[
            pltpu.VMEM((NCH, R), jnp.int32),
            pltpu.VMEM((NCH, R), jnp.int32),
            pltpu.VMEM((R, D), jnp.float32),
            pltpu.VMEM((R, D), jnp.float32),
            pltpu.VMEM((R, D), jnp.float32),
            pltpu.VMEM((R, D), jnp.float32),
            pltpu.VMEM((G, D), jnp.float32),
            pltpu.VMEM((NCH, R), jnp.float32),
            pltpu.SemaphoreType.DMA,
            pltpu.SemaphoreType.DMA,
            pltpu.SemaphoreType.DMA,
            pltpu.SemaphoreType.DMA,
            pltpu.SemaphoreType.DMA,
            pltpu.SemaphoreType.DMA,
            pltpu.SemaphoreType.DMA,
            pltpu.SemaphoreType.DMA,
        ],
    )(_sc_pred_kernel)
    src_idx = src_words.reshape(NW, NCH, R)
    trg_idx = trg_words.reshape(NW, NCH, R)
    return kern(src_idx, trg_idx, u_embeddings, v_embeddings)


def _loss_kernel(pred_ref, w_ref, y_ref, out_ref):
    p = pred_ref[...]
    w = w_ref[...]
    y = y_ref[...]
    bce = jnp.maximum(p, 0.0) - p * y + jnp.log1p(jnp.exp(-jnp.abs(p)))
    wl = w * bce
    num = jnp.sum(wl, axis=1)
    den = jnp.sum(w, axis=1)
    out_ref[0, 0] = jnp.mean(num / den)


def kernel(src_words, trg_words, wmasks, labels, u_embeddings, v_embeddings):
    pred = _sc_pred(src_words, trg_words, u_embeddings, v_embeddings)
    pred = pred.reshape(B, K)
    loss = pl.pallas_call(
        _loss_kernel,
        out_shape=jax.ShapeDtypeStruct((1, 1), jnp.float32),
        out_specs=pl.BlockSpec(memory_space=pltpu.SMEM),
    )(pred, wmasks, labels)
    return loss.reshape(())
